# Initial kernel scaffold; baseline (speedup 1.0000x reference)
import jax
import jax.numpy as jnp
from jax.experimental import pallas as pl


def kernel(x, edge_index, edge_weight, W_gcn, b_gcn, W_lin, b_lin):
    raise NotImplementedError("write your pallas kernel here")


# speedup vs baseline: 10.2441x; 10.2441x over previous
import functools

import jax
import jax.numpy as jnp
from jax import lax
from jax.experimental import pallas as pl
from jax.experimental.pallas import tpu as pltpu
from jax.experimental.pallas import tpu_sc as plsc

N = 10000
E = 160000
D = 256
DH = 128
NS = 16
L = 16
NPAD = 10240
RT = NPAD // NS
EPT = 10240
CR = 128
NR = EPT // CR
C = 64
NCH = EPT // C
NB = 2


def _mm_body(x_ref, w_ref, h0_ref, h1_ref):
    h = jnp.dot(x_ref[...], w_ref[...], preferred_element_type=jnp.float32)
    h0_ref[...] = h[:, :DH]
    h1_ref[...] = h[:, DH:]


def _matmul(x_pad, W_gcn):
    blk = 512
    return pl.pallas_call(
        _mm_body,
        grid=(NPAD // blk,),
        in_specs=[
            pl.BlockSpec((blk, D), lambda i: (i, 0)),
            pl.BlockSpec((D, D), lambda i: (0, 0)),
        ],
        out_specs=[pl.BlockSpec((blk, DH), lambda i: (i, 0))] * 2,
        out_shape=[jax.ShapeDtypeStruct((NPAD, DH), jnp.float32)] * 2,
    )(x_pad, W_gcn)


def _sc_agg(pk3, ew3, h0, h1):
    mesh = plsc.VectorSubcoreMesh(core_axis_name="c", subcore_axis_name="s")

    @functools.partial(
        pl.kernel,
        out_type=[jax.ShapeDtypeStruct((NPAD, DH), jnp.float32)] * 2,
        mesh=mesh,
        compiler_params=pltpu.CompilerParams(needs_layout_passes=False),
        scratch_types=[
            pltpu.VMEM_SHARED((NPAD, DH), jnp.float32),
            pltpu.VMEM_SHARED((NPAD,), jnp.float32),
            pltpu.VMEM((NR, CR), jnp.int32),
            pltpu.VMEM((NR, CR), jnp.float32),
            pltpu.VMEM((RT,), jnp.float32),
            pltpu.VMEM((CR,), jnp.int32),
            pltpu.VMEM((CR,), jnp.int32),
            pltpu.VMEM((CR,), jnp.float32),
            pltpu.VMEM((CR,), jnp.float32),
            pltpu.VMEM((NB, C), jnp.int32),
            pltpu.VMEM((C,), jnp.int32),
            pltpu.VMEM((C, DH), jnp.float32),
            pltpu.VMEM((C, DH), jnp.float32),
            pltpu.SemaphoreType.DMA,
            pltpu.SemaphoreType.DMA,
            pltpu.SemaphoreType.DMA,
        ],
    )
    def sc_kernel(pk_hbm, ew_hbm, h0_hbm, h1_hbm, agg0_hbm, agg1_hbm,
                  acc, deg, pkv, ewv, wbuf, rowb3, colb3, drb, dcb,
                  rowb, colb, gbuf0, gbuf1, sem0, sem1, sem2):
        c = lax.axis_index("c")
        s = lax.axis_index("s")
        rbase = s * RT
        m14 = jnp.full((L,), 0x3FFF, jnp.int32)

        pltpu.sync_copy(pk_hbm.at[s], pkv)
        pltpu.sync_copy(ew_hbm.at[s], ewv)

        @pl.loop(0, RT // L)
        def _(k):
            wbuf[pl.ds(k * L, L)] = jnp.full((L,), 1.0, jnp.float32)
        pltpu.sync_copy(wbuf, deg.at[pl.ds(rbase, RT)])
        plsc.subcore_barrier()

        @pl.loop(0, NR)
        def _(j):
            for i in range(CR // L):
                sl = pl.ds(i * L, L)
                colb3[sl] = lax.shift_right_logical(pkv[j, sl], 14)
            pltpu.sync_copy(ewv.at[j], deg.at[colb3], add=True)
        plsc.subcore_barrier()

        pltpu.sync_copy(deg.at[pl.ds(rbase, RT)], wbuf)

        @pl.loop(0, RT // L)
        def _(k):
            d = wbuf[pl.ds(k * L, L)]
            i = lax.bitcast_convert_type(d, jnp.int32)
            i = jnp.full((L,), 0x5F3759DF, jnp.int32) - lax.shift_right_logical(i, 1)
            y = lax.bitcast_convert_type(i, jnp.float32)
            for _ in range(3):
                y = y * (1.5 - 0.5 * d * y * y)
            wbuf[pl.ds(k * L, L)] = y
        pltpu.sync_copy(wbuf, deg.at[pl.ds(rbase, RT)])
        plsc.subcore_barrier()

        @pl.loop(0, NR)
        def _(j):
            for i in range(CR // L):
                sl = pl.ds(i * L, L)
                p = pkv[j, sl]
                rowb3[sl] = p & m14
                colb3[sl] = lax.shift_right_logical(p, 14)
            cp_r = pltpu.async_copy(deg.at[rowb3], drb, sem0)
            cp_c = pltpu.async_copy(deg.at[colb3], dcb, sem1)
            cp_r.wait()
            cp_c.wait()
            for i in range(CR // L):
                sl = pl.ds(i * L, L)
                ewv[j, sl] = drb[sl] * ewv[j, sl] * dcb[sl]

        def _pass(hk_hbm, agg_hbm):
            @pl.loop(0, RT // C)
            def _(t):
                base = rbase + t * C
                pltpu.sync_copy(hk_hbm.at[pl.ds(base, C)], gbuf0)

                @pl.loop(0, C // L)
                def _(i16):
                    dvec = wbuf[pl.ds(t * C + i16 * L, L)]
                    dvec = dvec * dvec
                    for k in range(L):
                        vv = jnp.full((L,), dvec[k], jnp.float32)
                        for jj in range(DH // L):
                            fsl = pl.ds(jj * L, L)
                            gbuf0[i16 * L + k, fsl] = gbuf0[i16 * L + k, fsl] * vv
                pltpu.sync_copy(gbuf0, acc.at[pl.ds(base, C)])
            plsc.subcore_barrier()

            bufs = (gbuf0, gbuf1)
            sems = (sem0, sem1)

            def unpack_row(j2, b):
                for i in range(C // L):
                    rowb[b, pl.ds(i * L, L)] = (
                        pkv[j2, pl.ds(b * C + i * L, L)] & m14)

            def unpack_col(j2, b):
                for i in range(C // L):
                    colb[pl.ds(i * L, L)] = lax.shift_right_logical(
                        pkv[j2, pl.ds(b * C + i * L, L)], 14)

            for b in range(NB):
                unpack_row(0, b)
                pltpu.async_copy(hk_hbm.at[rowb.at[b]], bufs[b], sems[b])

            @pl.loop(0, NCH, step=NB)
            def _(j0):
                j2 = j0 // 2
                for b in range(NB):
                    pltpu.make_async_copy(
                        hk_hbm.at[rowb.at[b]], bufs[b], sems[b]).wait()

                    @pl.loop(0, C // L)
                    def _(i16):
                        nvec = ewv[j2, pl.ds(b * C + i16 * L, L)]
                        for k in range(L):
                            vv = jnp.full((L,), nvec[k], jnp.float32)
                            for jj in range(DH // L):
                                fsl = pl.ds(jj * L, L)
                                bufs[b][i16 * L + k, fsl] = (
                                    bufs[b][i16 * L + k, fsl] * vv)

                    unpack_col(j2, b)
                    pltpu.sync_copy(bufs[b], acc.at[colb], add=True)

                    @pl.when(j0 + b + NB < NCH)
                    def _():
                        unpack_row(j2 + 1, b)
                        pltpu.async_copy(
                            hk_hbm.at[rowb.at[b]], bufs[b], sems[b])

            plsc.subcore_barrier()
            pltpu.sync_copy(acc.at[pl.ds(rbase, RT)],
                            agg_hbm.at[pl.ds(rbase, RT)])

        @pl.when(c == 0)
        def _():
            _pass(h0_hbm, agg0_hbm)

        @pl.when(c == 1)
        def _():
            _pass(h1_hbm, agg1_hbm)

    return sc_kernel(pk3, ew3, h0, h1)


def _head_body(a0, a1, b0, b1, w0, w1, bl, o_ref):
    acc = bl[...]
    for a, b, w in ((a0, b0, w0), (a1, b1, w1)):
        z = jnp.maximum(a[...] + b[...], 0.0)
        acc = acc + jnp.dot(z, w[...], preferred_element_type=jnp.float32)
    o_ref[...] = acc


def _head(aggs, b_gcn, W_lin, b_lin):
    blk = 400
    grid = N // blk
    bs = [b_gcn[i * DH:(i + 1) * DH].reshape(1, DH) for i in range(2)]
    ws = [W_lin[i * DH:(i + 1) * DH] for i in range(2)]
    bl = b_lin.reshape(1, 1)
    return pl.pallas_call(
        _head_body,
        grid=(grid,),
        in_specs=(
            [pl.BlockSpec((blk, DH), lambda i: (i, 0))] * 2
            + [pl.BlockSpec((1, DH), lambda i: (0, 0))] * 2
            + [pl.BlockSpec((DH, 1), lambda i: (0, 0))] * 2
            + [pl.BlockSpec((1, 1), lambda i: (0, 0))]
        ),
        out_specs=pl.BlockSpec((blk, 1), lambda i: (i, 0)),
        out_shape=jax.ShapeDtypeStruct((N, 1), jnp.float32),
    )(*aggs, *bs, *ws, bl)


def kernel(x, edge_index, edge_weight, W_gcn, b_gcn, W_lin, b_lin):
    f32 = jnp.float32
    row = edge_index[0].astype(jnp.int32)
    col = edge_index[1].astype(jnp.int32)
    ew = edge_weight.astype(f32)

    ept0 = E // NS
    pad = EPT - ept0
    pk = (col << 14) | row
    pk_t = pk.reshape(NS, ept0)
    ew_t = ew.reshape(NS, ept0)
    dummy = ((N + (jnp.arange(pad, dtype=jnp.int32) % (NPAD - N))) << 14)
    pk3 = jnp.concatenate(
        [pk_t, jnp.broadcast_to(dummy, (NS, pad))], axis=1
    ).reshape(NS, NR, CR)
    ew3 = jnp.pad(ew_t, ((0, 0), (0, pad))).reshape(NS, NR, CR)

    x_pad = jnp.pad(x.astype(f32), ((0, NPAD - N), (0, 0)))
    h0, h1 = _matmul(x_pad, W_gcn.astype(f32))
    aggs = _sc_agg(pk3, ew3, h0, h1)
    return _head(aggs, b_gcn.astype(f32), W_lin.astype(f32), b_lin.astype(f32))
